# baseline (speedup 1.0000x reference)
import jax
import jax.numpy as jnp
from jax import lax
from jax.experimental import pallas as pl
from jax.experimental.pallas import tpu as pltpu
from jax.experimental.pallas import tpu_sc as plsc

NUM_CORES = 2
NUM_SUBCORES = 16
NUM_WORKERS = NUM_CORES * NUM_SUBCORES
LANES = 16
BATCH = 16384
LATENT_DIM = 32
B_PER_W = BATCH // NUM_WORKERS
CHUNK = 128
N_CHUNKS = B_PER_W // CHUNK
N_GROUPS = B_PER_W // LANES


def _factorizer_body(uidx_hbm, cidx_hbm, utab_hbm, itab_hbm,
                     logit_hbm, score_hbm,
                     uidx_v, cidx_v, urows, irows, llog, lsco, sem):
    wid = lax.axis_index("s") * NUM_CORES + lax.axis_index("c")
    base = wid * B_PER_W

    pltpu.sync_copy(uidx_hbm.at[wid], uidx_v)
    pltpu.sync_copy(cidx_hbm.at[wid], cidx_v)

    copies = []
    for j in range(N_CHUNKS):
        copies.append(pltpu.async_copy(
            utab_hbm.at[uidx_v.at[j]], urows.at[pl.ds(j * CHUNK, CHUNK)], sem))
        copies.append(pltpu.async_copy(
            itab_hbm.at[cidx_v.at[j]], irows.at[pl.ds(j * CHUNK, CHUNK)], sem))
    for cp in copies:
        cp.wait()

    lane = lax.iota(jnp.int32, 16)

    def group(g, carry):
        ridx = g * LANES + lane
        acc = jnp.zeros((16,), jnp.float32)
        for d in range(LATENT_DIM):
            dv = jnp.full((16,), d, jnp.int32)
            acc = acc + (plsc.load_gather(urows, [ridx, dv]) *
                         plsc.load_gather(irows, [ridx, dv]))
        llog[pl.ds(g * LANES, LANES)] = acc
        lsco[pl.ds(g * LANES, LANES)] = 1.0 / (1.0 + jnp.exp(-acc))
        return carry

    lax.fori_loop(0, N_GROUPS, group, 0)

    pltpu.sync_copy(llog, logit_hbm.at[pl.ds(base, B_PER_W)])
    pltpu.sync_copy(lsco, score_hbm.at[pl.ds(base, B_PER_W)])


@jax.jit
def kernel(user_ids, content_ids, user_matrix, item_matrix):
    uidx = user_ids.astype(jnp.int32).reshape(NUM_WORKERS, N_CHUNKS, CHUNK)
    cidx = content_ids.astype(jnp.int32).reshape(NUM_WORKERS, N_CHUNKS, CHUNK)

    run = pl.kernel(
        _factorizer_body,
        out_type=(
            jax.ShapeDtypeStruct((BATCH,), jnp.float32),
            jax.ShapeDtypeStruct((BATCH,), jnp.float32),
        ),
        mesh=plsc.VectorSubcoreMesh(core_axis_name="c", subcore_axis_name="s"),
        compiler_params=pltpu.CompilerParams(
            needs_layout_passes=False, use_tc_tiling_on_sc=False),
        scratch_types=[
            pltpu.VMEM((N_CHUNKS, CHUNK), jnp.int32),
            pltpu.VMEM((N_CHUNKS, CHUNK), jnp.int32),
            pltpu.VMEM((B_PER_W, LATENT_DIM), jnp.float32),
            pltpu.VMEM((B_PER_W, LATENT_DIM), jnp.float32),
            pltpu.VMEM((B_PER_W,), jnp.float32),
            pltpu.VMEM((B_PER_W,), jnp.float32),
            pltpu.SemaphoreType.DMA,
        ],
    )
    logits, scores = run(uidx, cidx, user_matrix, item_matrix)
    return (logits[:, None], scores[:, None])
